# baseline (speedup 1.0000x reference)
import jax
import jax.numpy as jnp
from jax.experimental import pallas as pl
from jax.experimental.pallas import tpu as pltpu

_N = 20000
_D = 1024
_OUT_B = 320
_BM = 1000


def _fused_heads(x_ref, wb_ref, bb_ref, wi_ref, bi_ref, ob_ref, oi_ref):
    x = x_ref[...].astype(jnp.bfloat16)
    ob_ref[...] = (
        jnp.dot(
            x, wb_ref[...].astype(jnp.bfloat16), preferred_element_type=jnp.float32
        )
        + bb_ref[...]
    )
    oi_ref[...] = (
        jnp.dot(
            x, wi_ref[...].astype(jnp.bfloat16), preferred_element_type=jnp.float32
        )
        + bi_ref[...]
    )


def kernel(x, W_bbox, b_bbox, W_iou, b_iou):
    if x.ndim > 2:
        x = x.reshape(x.shape[0], -1)
    n, d = x.shape
    out_b = W_bbox.shape[1]
    bb2 = b_bbox.reshape(1, out_b)
    bi2 = b_iou.reshape(1, 1)

    grid = (n // _BM,)
    deltas, iou = pl.pallas_call(
        _fused_heads,
        grid=grid,
        in_specs=[
            pl.BlockSpec((_BM, d), lambda i: (i, 0)),
            pl.BlockSpec((d, out_b), lambda i: (0, 0)),
            pl.BlockSpec((1, out_b), lambda i: (0, 0)),
            pl.BlockSpec((d, 1), lambda i: (0, 0)),
            pl.BlockSpec((1, 1), lambda i: (0, 0)),
        ],
        out_specs=[
            pl.BlockSpec((_BM, out_b), lambda i: (i, 0)),
            pl.BlockSpec((_BM, 1), lambda i: (i, 0)),
        ],
        out_shape=[
            jax.ShapeDtypeStruct((n, out_b), jnp.float32),
            jax.ShapeDtypeStruct((n, 1), jnp.float32),
        ],
        compiler_params=pltpu.CompilerParams(
            dimension_semantics=("arbitrary",),
        ),
    )(x, W_bbox, bb2, W_iou, bi2)
    return (deltas, iou)


# speedup vs baseline: 1.0249x; 1.0249x over previous
import jax
import jax.numpy as jnp
from jax.experimental import pallas as pl
from jax.experimental.pallas import tpu as pltpu

_N = 20000
_D = 1024
_OUT_B = 320
_BM = 1000


def _fused_heads(x_ref, wb_ref, bb_ref, wi_ref, bi_ref, ob_ref, oi_ref):
    x = x_ref[...].astype(jnp.bfloat16)
    ob_ref[...] = (
        jnp.dot(
            x, wb_ref[...].astype(jnp.bfloat16), preferred_element_type=jnp.float32
        )
        + bb_ref[...]
    )
    oi_ref[...] = (
        jnp.dot(
            x, wi_ref[...].astype(jnp.bfloat16), preferred_element_type=jnp.float32
        )
        + bi_ref[...]
    )


def kernel(x, W_bbox, b_bbox, W_iou, b_iou):
    if x.ndim > 2:
        x = x.reshape(x.shape[0], -1)
    n, d = x.shape
    out_b = W_bbox.shape[1]
    bb2 = b_bbox.reshape(1, out_b)
    bi2 = b_iou.reshape(1, 1)

    grid = (n // _BM,)
    deltas, iou = pl.pallas_call(
        _fused_heads,
        grid=grid,
        in_specs=[
            pl.BlockSpec((_BM, d), lambda i: (i, 0)),
            pl.BlockSpec((d, out_b), lambda i: (0, 0)),
            pl.BlockSpec((1, out_b), lambda i: (0, 0)),
            pl.BlockSpec((d, 1), lambda i: (0, 0)),
            pl.BlockSpec((1, 1), lambda i: (0, 0)),
        ],
        out_specs=[
            pl.BlockSpec((_BM, out_b), lambda i: (i, 0)),
            pl.BlockSpec((_BM, 1), lambda i: (i, 0)),
        ],
        out_shape=[
            jax.ShapeDtypeStruct((n, out_b), jnp.float32),
            jax.ShapeDtypeStruct((n, 1), jnp.float32),
        ],
        compiler_params=pltpu.CompilerParams(
            dimension_semantics=("parallel",),
        ),
    )(x, W_bbox, bb2, W_iou, bi2)
    return (deltas, iou)


# speedup vs baseline: 1.0612x; 1.0355x over previous
import jax
import jax.numpy as jnp
from jax.experimental import pallas as pl
from jax.experimental.pallas import tpu as pltpu

_N = 20000
_D = 1024
_OUT_B = 320
_BM = 2000


def _fused_heads(x_ref, wb_ref, bb_ref, wi_ref, bi_ref, ob_ref, oi_ref):
    x = x_ref[...].astype(jnp.bfloat16)
    ob_ref[...] = (
        jnp.dot(
            x, wb_ref[...].astype(jnp.bfloat16), preferred_element_type=jnp.float32
        )
        + bb_ref[...]
    )
    oi_ref[...] = (
        jnp.dot(
            x, wi_ref[...].astype(jnp.bfloat16), preferred_element_type=jnp.float32
        )
        + bi_ref[...]
    )


def kernel(x, W_bbox, b_bbox, W_iou, b_iou):
    if x.ndim > 2:
        x = x.reshape(x.shape[0], -1)
    n, d = x.shape
    out_b = W_bbox.shape[1]
    bb2 = b_bbox.reshape(1, out_b)
    bi2 = b_iou.reshape(1, 1)

    grid = (n // _BM,)
    deltas, iou = pl.pallas_call(
        _fused_heads,
        grid=grid,
        in_specs=[
            pl.BlockSpec((_BM, d), lambda i: (i, 0)),
            pl.BlockSpec((d, out_b), lambda i: (0, 0)),
            pl.BlockSpec((1, out_b), lambda i: (0, 0)),
            pl.BlockSpec((d, 1), lambda i: (0, 0)),
            pl.BlockSpec((1, 1), lambda i: (0, 0)),
        ],
        out_specs=[
            pl.BlockSpec((_BM, out_b), lambda i: (i, 0)),
            pl.BlockSpec((_BM, 1), lambda i: (i, 0)),
        ],
        out_shape=[
            jax.ShapeDtypeStruct((n, out_b), jnp.float32),
            jax.ShapeDtypeStruct((n, 1), jnp.float32),
        ],
        compiler_params=pltpu.CompilerParams(
            dimension_semantics=("parallel",),
        ),
    )(x, W_bbox, bb2, W_iou, bi2)
    return (deltas, iou)


# speedup vs baseline: 1.0898x; 1.0270x over previous
import jax
import jax.numpy as jnp
from jax.experimental import pallas as pl
from jax.experimental.pallas import tpu as pltpu

_BM = 2000
_SPLIT = 2


def _fused_heads(*refs):
    x_refs = refs[:_SPLIT]
    wb_ref, bb_ref, wi_ref, bi_ref, ob_ref, oi_ref = refs[_SPLIT:]
    wb = wb_ref[...].astype(jnp.bfloat16)
    wi = wi_ref[...].astype(jnp.bfloat16)
    bb = bb_ref[...]
    bi = bi_ref[...]
    half = _BM // _SPLIT
    for j, x_ref in enumerate(x_refs):
        x = x_ref[...].astype(jnp.bfloat16)
        rows = pl.ds(j * half, half)
        ob_ref[rows, :] = jnp.dot(x, wb, preferred_element_type=jnp.float32) + bb
        oi_ref[rows, :] = jnp.dot(x, wi, preferred_element_type=jnp.float32) + bi


def kernel(x, W_bbox, b_bbox, W_iou, b_iou):
    if x.ndim > 2:
        x = x.reshape(x.shape[0], -1)
    n, d = x.shape
    out_b = W_bbox.shape[1]
    bb2 = b_bbox.reshape(1, out_b)
    bi2 = b_iou.reshape(1, 1)

    half = _BM // _SPLIT
    grid = (n // _BM,)

    def row_idx(j):
        return lambda i: (_SPLIT * i + j, 0)

    x_specs = [pl.BlockSpec((half, d), row_idx(j)) for j in range(_SPLIT)]

    deltas, iou = pl.pallas_call(
        _fused_heads,
        grid=grid,
        in_specs=x_specs + [
            pl.BlockSpec((d, out_b), lambda i: (0, 0)),
            pl.BlockSpec((1, out_b), lambda i: (0, 0)),
            pl.BlockSpec((d, 1), lambda i: (0, 0)),
            pl.BlockSpec((1, 1), lambda i: (0, 0)),
        ],
        out_specs=[
            pl.BlockSpec((_BM, out_b), lambda i: (i, 0)),
            pl.BlockSpec((_BM, 1), lambda i: (i, 0)),
        ],
        out_shape=[
            jax.ShapeDtypeStruct((n, out_b), jnp.float32),
            jax.ShapeDtypeStruct((n, 1), jnp.float32),
        ],
        compiler_params=pltpu.CompilerParams(
            dimension_semantics=("parallel",),
        ),
    )(*([x] * _SPLIT + [W_bbox, bb2, W_iou, bi2]))
    return (deltas, iou)


# speedup vs baseline: 1.1436x; 1.0494x over previous
import jax
import jax.numpy as jnp
from jax.experimental import pallas as pl
from jax.experimental.pallas import tpu as pltpu

_BM = 1000
_NBUF = 4


def _fused_heads(x_hbm, wb_ref, bb_ref, wi_ref, bi_ref, ob_ref, oi_ref,
                 xbuf, sems):
    i = pl.program_id(0)
    nsteps = pl.num_programs(0)

    def start_copy(step):
        slot = jax.lax.rem(step, _NBUF)
        pltpu.make_async_copy(
            x_hbm.at[pl.ds(step * _BM, _BM), :],
            xbuf.at[slot],
            sems.at[slot],
        ).start()

    @pl.when(i == 0)
    def _prologue():
        for k in range(_NBUF - 1):
            start_copy(k)

    nxt = i + _NBUF - 1

    @pl.when(nxt < nsteps)
    def _refill():
        start_copy(nxt)

    slot = jax.lax.rem(i, _NBUF)
    pltpu.make_async_copy(
        x_hbm.at[pl.ds(i * _BM, _BM), :],
        xbuf.at[slot],
        sems.at[slot],
    ).wait()

    x = xbuf[slot].astype(jnp.bfloat16)
    wb = wb_ref[...].astype(jnp.bfloat16)
    wi = wi_ref[...].astype(jnp.bfloat16)
    ob_ref[...] = jnp.dot(x, wb, preferred_element_type=jnp.float32) + bb_ref[...]
    oi_ref[...] = jnp.dot(x, wi, preferred_element_type=jnp.float32) + bi_ref[...]


def kernel(x, W_bbox, b_bbox, W_iou, b_iou):
    if x.ndim > 2:
        x = x.reshape(x.shape[0], -1)
    n, d = x.shape
    out_b = W_bbox.shape[1]
    bb2 = b_bbox.reshape(1, out_b)
    bi2 = b_iou.reshape(1, 1)

    grid = (n // _BM,)
    deltas, iou = pl.pallas_call(
        _fused_heads,
        grid=grid,
        in_specs=[
            pl.BlockSpec(memory_space=pltpu.MemorySpace.HBM),
            pl.BlockSpec((d, out_b), lambda i: (0, 0)),
            pl.BlockSpec((1, out_b), lambda i: (0, 0)),
            pl.BlockSpec((d, 1), lambda i: (0, 0)),
            pl.BlockSpec((1, 1), lambda i: (0, 0)),
        ],
        out_specs=[
            pl.BlockSpec((_BM, out_b), lambda i: (i, 0)),
            pl.BlockSpec((_BM, 1), lambda i: (i, 0)),
        ],
        out_shape=[
            jax.ShapeDtypeStruct((n, out_b), jnp.float32),
            jax.ShapeDtypeStruct((n, 1), jnp.float32),
        ],
        scratch_shapes=[
            pltpu.VMEM((_NBUF, _BM, d), jnp.float32),
            pltpu.SemaphoreType.DMA((_NBUF,)),
        ],
        compiler_params=pltpu.CompilerParams(
            dimension_semantics=("arbitrary",),
        ),
    )(x, W_bbox, bb2, W_iou, bi2)
    return (deltas, iou)
